# baseline (speedup 1.0000x reference)
import functools
import math

import jax
import jax.numpy as jnp
import numpy as np
from jax import lax
from jax.experimental import pallas as pl
from jax.experimental.pallas import tpu as pltpu

_B, _T, _N, _D = 4, 12, 1024, 128
_C_FRONT = 78
_X_LAST = _T // 2 - 1
_BS = 6
_SIGMA = 10.0
_LN_EPS = 1e-5
_INTERPRET = False


def _attn_kernel(inp_ref, gt_ref, w_ref, b_ref, vi_ref, rs_ref, y_ref, ws_ref):
    f32 = jnp.float32
    x = jnp.dot(inp_ref[0], w_ref[...], preferred_element_type=f32) + b_ref[...]
    q = x[:, :_D]
    k = x[:, _D:2 * _D]
    v = x[:, 2 * _D:]
    gtk = lax.dot_general(gt_ref[...], k, (((0,), (0,)), ((), ())),
                          preferred_element_type=f32)
    e2 = lax.dot_general(q, gtk, (((1,), (1,)), ((), ())),
                         preferred_element_type=f32)
    rowmax = jnp.max(e2, axis=1, keepdims=True)
    p = jnp.exp(e2 - rowmax)
    rowsum = jnp.sum(p, axis=1, keepdims=True)
    inv = 1.0 / rowsum
    pv = jnp.dot(p, v, preferred_element_type=f32)
    vi_ref[0] = pv * inv
    rs_ref[0] = jnp.concatenate(
        [rowmax.T, rowsum.T], axis=0)
    bt = pl.program_id(0) * (_B * _T // 2) + pl.program_id(1)

    @pl.when(bt % _T == _X_LAST)
    def _():
        cs = jnp.sum(p * inv, axis=0, keepdims=True)
        iota = lax.broadcasted_iota(jnp.int32, (1, _N), 1)
        m1 = jnp.min(cs, axis=1, keepdims=True)
        i1 = jnp.min(jnp.where(cs == m1, iota, _N), axis=1, keepdims=True)
        masked = jnp.where(iota == i1, jnp.float32(jnp.inf), cs)
        m2 = jnp.min(masked, axis=1, keepdims=True)
        y2 = jnp.min(jnp.where(masked == m2, iota, _N), axis=1, keepdims=True)
        yv = y2[0, 0]
        y_ref[0, 0, 0] = yv
        ws_ref[0, 0, 0] = jnp.clip(yv - 2 * _BS, 0, _N - _W)


def _ffn_kernel(inp_ref, vi_ref, rs_ref, gct_ref, delta_ref,
                wqk_ref, bqk_ref, f1_ref, b1_ref, f2_ref, b2_ref,
                lnw_ref, lnb_ref, out_ref):
    f32 = jnp.float32
    inp = inp_ref[0, 0]
    x = jnp.dot(inp, wqk_ref[...], preferred_element_type=f32) + bqk_ref[...]
    q = x[:, :_D]
    k = x[:, _D:]
    mt = lax.dot_general(gct_ref[0], k, (((0,), (0,)), ((), ())),
                         preferred_element_type=f32)
    ecT = lax.dot_general(mt, q, (((1,), (1,)), ((), ())),
                          preferred_element_type=f32)
    rowmaxT = rs_ref[0, 0, 0:1, :]
    rowsumT = rs_ref[0, 0, 1:2, :]
    acT = jnp.exp(ecT - rowmaxT) / rowsumT
    corr = lax.dot_general(acT, delta_ref[0, 0], (((0,), (0,)), ((), ())),
                           preferred_element_type=f32)
    vi = vi_ref[0, 0] + corr
    h = jnp.dot(vi, f1_ref[...], preferred_element_type=f32) + b1_ref[...]
    h = 0.5 * h * (1.0 + lax.erf(h * (1.0 / math.sqrt(2.0))))
    h = jnp.dot(h, f2_ref[...], preferred_element_type=f32) + b2_ref[...]
    out = h + inp
    mu = jnp.mean(out, axis=1, keepdims=True)
    cent = out - mu
    var = jnp.mean(cent * cent, axis=1, keepdims=True)
    out_ref[0, 0] = (cent / jnp.sqrt(var + _LN_EPS)) * lnw_ref[...] + lnb_ref[...]


_WG_XS = np.arange(_X_LAST, dtype=np.float32)
_WG = np.exp(-((_WG_XS - np.float32(_X_LAST - 1.0)) ** 2)
             / np.float32(2.0 * _SIGMA ** 2)).astype(np.float32)
_WG = (_WG / _WG.sum()).tolist()


def _win_kernel(ws_ref, y_ref, inpa_ref, inpb_ref, wv_ref, bv_ref,
                mixp_ref, orig_ref, xs_ref, wpad_ref):
    f32 = jnp.float32
    b = pl.program_id(0)
    ws = ws_ref[b]
    y = y_ref[b]
    c0 = jnp.minimum(ws // 128, _N // 128 - 2)
    off = ws - c0 * 128
    xs_ref[:, 0:128, :] = inpa_ref[0]
    xs_ref[:, 128:256, :] = inpb_ref[0]
    sl = xs_ref[:, pl.ds(off, _W), :]
    v_win = jnp.dot(sl.reshape(_T * _W, _D), wv_ref[...],
                    preferred_element_type=f32) + bv_ref[...]
    wpad_ref[:, 0:_W, :] = v_win.reshape(_T, _W, _D)
    wpad_ref[:, _W:2 * _W, :] = jnp.zeros((_T, _W, _D), f32)

    yl = y - ws
    y_start = jnp.maximum(y - _BS, 0)
    y_end = jnp.minimum(y + 2 * _BS, _N)
    coly = wpad_ref[:, pl.ds(yl, 1), :][:, 0, :]
    mv = jnp.zeros((1, _D), f32)
    for i in range(_X_LAST):
        mv = mv + _WG[i] * coly[i:i + 1, :]
    tmask = lax.broadcasted_iota(jnp.int32, (_T, 1, 1), 0) == _X_LAST

    j0 = y_start // _BS
    j1 = y_end // _BS
    cnt_a = j1 - j0
    l0 = j0 * _BS - ws
    blk_a = wpad_ref[:, pl.ds(l0, 4 * _BS), :]
    cm_a = lax.broadcasted_iota(jnp.int32, (1, 4 * _BS, 1), 1) == (y - j0 * _BS)
    blk_a = jnp.where(tmask & cm_a, mv[:, None, :], blk_a)
    acc = jnp.zeros((_T, _BS, _D), f32)
    for j in range(4):
        wj = (j < cnt_a).astype(f32)
        acc = acc + wj * blk_a[:, j * _BS:(j + 1) * _BS, :]
    mixed_a = jnp.concatenate(
        [acc / cnt_a.astype(f32), jnp.zeros((_T, 2, _D), f32)], axis=1)
    orig_a = wpad_ref[:, pl.ds(yl, 8), :]

    cnt_b = y_end - y_start
    lb = y_start - ws
    blk_b = wpad_ref[:, pl.ds(lb, 3 * _BS), :]
    cm_b = lax.broadcasted_iota(jnp.int32, (1, 3 * _BS, 1), 1) == (y - y_start)
    blk_b = jnp.where(tmask & cm_b, mv[:, None, :], blk_b)
    mj_b = (lax.broadcasted_iota(jnp.int32, (1, 3 * _BS, 1), 1)
            < cnt_b).astype(f32)
    mixed_b = (blk_b * mj_b).sum(1) / cnt_b.astype(f32)
    off_b = y - jnp.minimum(y, _N - 8)
    io8 = lax.broadcasted_iota(jnp.int32, (1, 8, 1), 1) == off_b
    mixed_b8 = jnp.where(io8, mixed_b[:, None, :], 0.0)
    orig_b8 = jnp.where(io8, coly[:, None, :], 0.0)

    is_b = y_end > _N - _BS
    mixp_ref[0] = jnp.where(is_b, mixed_b8, mixed_a)
    orig_ref[0] = jnp.where(is_b, orig_b8, orig_a)


_W = 32


def _bt4(c, j):
    bt = c * (_B * _T // 2) + j
    return (bt // _T, bt % _T, 0, 0)


def kernel(input, adapt_G, xff_w, xff_b, ff1_w, ff1_b, ff2_w, ff2_b, ln_w, ln_b):
    f32 = jnp.float32
    B, T, N, D = _B, _T, _N, _D
    BT = B * T
    scale = 1.0 / math.sqrt(D)

    inp3 = input.reshape(BT, N, D)
    gT = adapt_G
    qscale = jnp.concatenate(
        [jnp.full((D,), scale, f32), jnp.ones((2 * D,), f32)])
    xffT_s = xff_w.T * qscale[None, :]
    xffb_s = (xff_b * qscale)[None, :]

    vi_raw, rowstats, y_arr, ws_arr = pl.pallas_call(
        _attn_kernel,
        grid=(2, BT // 2),
        in_specs=[
            pl.BlockSpec((1, N, D), lambda c, j: (c * (BT // 2) + j, 0, 0)),
            pl.BlockSpec((N, N), lambda c, j: (0, 0)),
            pl.BlockSpec((D, 3 * D), lambda c, j: (0, 0)),
            pl.BlockSpec((1, 3 * D), lambda c, j: (0, 0)),
        ],
        out_specs=[
            pl.BlockSpec((1, N, D), lambda c, j: (c * (BT // 2) + j, 0, 0)),
            pl.BlockSpec((1, 2, N), lambda c, j: (c * (BT // 2) + j, 0, 0)),
            pl.BlockSpec((1, 1, 1), lambda c, j: ((c * (BT // 2) + j) // T, 0, 0),
                         memory_space=pltpu.SMEM),
            pl.BlockSpec((1, 1, 1), lambda c, j: ((c * (BT // 2) + j) // T, 0, 0),
                         memory_space=pltpu.SMEM),
        ],
        out_shape=[
            jax.ShapeDtypeStruct((BT, N, D), f32),
            jax.ShapeDtypeStruct((BT, 2, N), f32),
            jax.ShapeDtypeStruct((B, 1, 1), jnp.int32),
            jax.ShapeDtypeStruct((B, 1, 1), jnp.int32),
        ],
        compiler_params=pltpu.CompilerParams(
            dimension_semantics=("parallel", "arbitrary"),
            vmem_limit_bytes=56 * 1024 * 1024,
        ),
        name="attn_fused",
        interpret=_INTERPRET,
    )(inp3, gT, xffT_s, xffb_s)

    y = y_arr.reshape(B)
    ws = ws_arr.reshape(B)
    inp4 = input.reshape(B, T, N, D)
    nb = N // 128
    mixp, orig = pl.pallas_call(
        _win_kernel,
        grid_spec=pltpu.PrefetchScalarGridSpec(
            num_scalar_prefetch=2,
            grid=(B,),
            in_specs=[
                pl.BlockSpec(
                    (1, T, 128, D),
                    lambda b, ws_r, y_r: (b, 0, jnp.minimum(ws_r[b] // 128, nb - 2), 0)),
                pl.BlockSpec(
                    (1, T, 128, D),
                    lambda b, ws_r, y_r: (b, 0, jnp.minimum(ws_r[b] // 128, nb - 2) + 1, 0)),
                pl.BlockSpec((D, D), lambda b, ws_r, y_r: (0, 0)),
                pl.BlockSpec((1, D), lambda b, ws_r, y_r: (0, 0)),
            ],
            out_specs=[
                pl.BlockSpec((1, T, 8, D), lambda b, ws_r, y_r: (b, 0, 0, 0)),
                pl.BlockSpec((1, T, 8, D), lambda b, ws_r, y_r: (b, 0, 0, 0)),
            ],
            scratch_shapes=[pltpu.VMEM((T, 256, D), f32),
                            pltpu.VMEM((T, 2 * _W, D), f32)],
        ),
        out_shape=[jax.ShapeDtypeStruct((B, T, 8, D), f32),
                   jax.ShapeDtypeStruct((B, T, 8, D), f32)],
        name="v_window_mixup",
        interpret=_INTERPRET,
    )(ws, y, inp4, inp4, xffT_s[:, 2 * D:], xffb_s[:, 2 * D:])

    is_b = jnp.minimum(y + 2 * _BS, N) > N - _BS
    cs_arr = jnp.where(is_b, jnp.minimum(y, N - 8), y)
    act = jnp.where(is_b[:, None],
                    jnp.arange(8)[None, :] == (y - cs_arr)[:, None],
                    jnp.arange(8)[None, :] < _BS)
    mix_u8 = lax.stop_gradient(mixp.astype(jnp.uint8).astype(f32))
    delta = jnp.where(act[:, None, :, None]
                      & (jnp.arange(D) < _C_FRONT)[None, None, None, :],
                      mix_u8 - orig, 0.0)
    gcol = jax.vmap(
        lambda c: lax.dynamic_slice(adapt_G, (0, c), (N, 8)))(cs_arr)

    wqkT_s = xffT_s[:, :2 * D]
    bqk_s = xffb_s[:, :2 * D]

    out = pl.pallas_call(
        _ffn_kernel,
        grid=(2, BT // 2),
        in_specs=[
            pl.BlockSpec((1, 1, N, D), _bt4),
            pl.BlockSpec((1, 1, N, D), _bt4),
            pl.BlockSpec((1, 1, 2, N), _bt4),
            pl.BlockSpec((1, N, 8), lambda c, j: ((c * (BT // 2) + j) // T, 0, 0)),
            pl.BlockSpec((1, 1, 8, D), _bt4),
            pl.BlockSpec((D, 2 * D), lambda c, j: (0, 0)),
            pl.BlockSpec((1, 2 * D), lambda c, j: (0, 0)),
            pl.BlockSpec((D, D), lambda c, j: (0, 0)),
            pl.BlockSpec((1, D), lambda c, j: (0, 0)),
            pl.BlockSpec((D, D), lambda c, j: (0, 0)),
            pl.BlockSpec((1, D), lambda c, j: (0, 0)),
            pl.BlockSpec((1, D), lambda c, j: (0, 0)),
            pl.BlockSpec((1, D), lambda c, j: (0, 0)),
        ],
        out_specs=pl.BlockSpec((1, 1, N, D), _bt4),
        out_shape=jax.ShapeDtypeStruct((B, T, N, D), f32),
        compiler_params=pltpu.CompilerParams(
            dimension_semantics=("parallel", "arbitrary"),
        ),
        name="mixup_ffn_ln",
        interpret=_INTERPRET,
    )(input.reshape(B, T, N, D), vi_raw.reshape(B, T, N, D),
      rowstats.reshape(B, T, 2, N), gcol, delta,
      wqkT_s, bqk_s, ff1_w.T, ff1_b[None, :], ff2_w.T, ff2_b[None, :],
      ln_w[None, :], ln_b[None, :])

    return out


# speedup vs baseline: 1.0955x; 1.0955x over previous
import functools
import math

import jax
import jax.numpy as jnp
import numpy as np
from jax import lax
from jax.experimental import pallas as pl
from jax.experimental.pallas import tpu as pltpu

_B, _T, _N, _D = 4, 12, 1024, 128
_C_FRONT = 78
_X_LAST = _T // 2 - 1
_BS = 6
_SIGMA = 10.0
_LN_EPS = 1e-5
_INTERPRET = False


def _attn_kernel(inp_ref, gt_ref, w_ref, b_ref, vi_ref, rs_ref, y_ref, ws_ref):
    f32 = jnp.float32
    x = jnp.dot(inp_ref[0], w_ref[...], preferred_element_type=f32) + b_ref[...]
    q = x[:, :_D]
    k = x[:, _D:2 * _D]
    v = x[:, 2 * _D:]
    gtk = lax.dot_general(gt_ref[...], k, (((0,), (0,)), ((), ())),
                          preferred_element_type=f32)
    e2 = lax.dot_general(q, gtk, (((1,), (1,)), ((), ())),
                         preferred_element_type=f32)
    rowmax = jnp.max(e2, axis=1, keepdims=True)
    p = jnp.exp(e2 - rowmax)
    rowsum = jnp.sum(p, axis=1, keepdims=True)
    inv = 1.0 / rowsum
    pv = jnp.dot(p, v, preferred_element_type=f32)
    vi_ref[0] = pv * inv
    rs_ref[0] = jnp.concatenate(
        [rowmax.T, rowsum.T], axis=0)
    bt = pl.program_id(0) * (_B * _T // 2) + pl.program_id(1)

    @pl.when(bt % _T == _X_LAST)
    def _():
        cs = jnp.sum(p * inv, axis=0, keepdims=True)
        iota = lax.broadcasted_iota(jnp.int32, (1, _N), 1)
        m1 = jnp.min(cs, axis=1, keepdims=True)
        i1 = jnp.min(jnp.where(cs == m1, iota, _N), axis=1, keepdims=True)
        masked = jnp.where(iota == i1, jnp.float32(jnp.inf), cs)
        m2 = jnp.min(masked, axis=1, keepdims=True)
        y2 = jnp.min(jnp.where(masked == m2, iota, _N), axis=1, keepdims=True)
        yv = y2[0, 0]
        y_ref[0, 0, 0] = yv
        ws_ref[0, 0, 0] = jnp.clip(yv - 2 * _BS, 0, _N - _W)


_NB = 6


def _ffn_kernel(inp_ref, vi_ref, rs_ref, gct_ref, delta_ref,
                wqk_ref, bqk_ref, f1_ref, b1_ref, f2_ref, b2_ref,
                lnw_ref, lnb_ref, out_ref):
    f32 = jnp.float32
    inp = inp_ref[...].reshape(_NB * _N, _D)
    x = jnp.dot(inp, wqk_ref[...], preferred_element_type=f32) + bqk_ref[...]
    corrs = []
    for i in range(_NB):
        q = x[i * _N:(i + 1) * _N, :_D]
        k = x[i * _N:(i + 1) * _N, _D:]
        mt = lax.dot_general(gct_ref[0], k, (((0,), (0,)), ((), ())),
                             preferred_element_type=f32)
        ecT = lax.dot_general(mt, q, (((1,), (1,)), ((), ())),
                              preferred_element_type=f32)
        acT = jnp.exp(ecT - rs_ref[i, 0:1, :]) / rs_ref[i, 1:2, :]
        corrs.append(
            lax.dot_general(acT, delta_ref[i], (((0,), (0,)), ((), ())),
                            preferred_element_type=f32))
    vi = vi_ref[...].reshape(_NB * _N, _D) + jnp.concatenate(corrs, axis=0)
    h = jnp.dot(vi, f1_ref[...], preferred_element_type=f32) + b1_ref[...]
    h = 0.5 * h * (1.0 + lax.erf(h * (1.0 / math.sqrt(2.0))))
    h = jnp.dot(h, f2_ref[...], preferred_element_type=f32) + b2_ref[...]
    out = h + inp
    mu = jnp.mean(out, axis=1, keepdims=True)
    cent = out - mu
    var = jnp.mean(cent * cent, axis=1, keepdims=True)
    res = (cent / jnp.sqrt(var + _LN_EPS)) * lnw_ref[...] + lnb_ref[...]
    out_ref[...] = res.reshape(_NB, _N, _D)


_WG_XS = np.arange(_X_LAST, dtype=np.float32)
_WG = np.exp(-((_WG_XS - np.float32(_X_LAST - 1.0)) ** 2)
             / np.float32(2.0 * _SIGMA ** 2)).astype(np.float32)
_WG = (_WG / _WG.sum()).tolist()


def _win_kernel(ws_ref, y_ref, inpa_ref, inpb_ref, wv_ref, bv_ref,
                mixp_ref, orig_ref, xs_ref, wpad_ref):
    f32 = jnp.float32
    b = pl.program_id(0)
    ws = ws_ref[b]
    y = y_ref[b]
    c0 = jnp.minimum(ws // 128, _N // 128 - 2)
    off = ws - c0 * 128
    xs_ref[:, 0:128, :] = inpa_ref[0]
    xs_ref[:, 128:256, :] = inpb_ref[0]
    sl = xs_ref[:, pl.ds(off, _W), :]
    v_win = jnp.dot(sl.reshape(_T * _W, _D), wv_ref[...],
                    preferred_element_type=f32) + bv_ref[...]
    wpad_ref[:, 0:_W, :] = v_win.reshape(_T, _W, _D)
    wpad_ref[:, _W:2 * _W, :] = jnp.zeros((_T, _W, _D), f32)

    yl = y - ws
    y_start = jnp.maximum(y - _BS, 0)
    y_end = jnp.minimum(y + 2 * _BS, _N)
    coly = wpad_ref[:, pl.ds(yl, 1), :][:, 0, :]
    mv = jnp.zeros((1, _D), f32)
    for i in range(_X_LAST):
        mv = mv + _WG[i] * coly[i:i + 1, :]
    tmask = lax.broadcasted_iota(jnp.int32, (_T, 1, 1), 0) == _X_LAST

    j0 = y_start // _BS
    j1 = y_end // _BS
    cnt_a = j1 - j0
    l0 = j0 * _BS - ws
    blk_a = wpad_ref[:, pl.ds(l0, 4 * _BS), :]
    cm_a = lax.broadcasted_iota(jnp.int32, (1, 4 * _BS, 1), 1) == (y - j0 * _BS)
    blk_a = jnp.where(tmask & cm_a, mv[:, None, :], blk_a)
    acc = jnp.zeros((_T, _BS, _D), f32)
    for j in range(4):
        wj = (j < cnt_a).astype(f32)
        acc = acc + wj * blk_a[:, j * _BS:(j + 1) * _BS, :]
    mixed_a = jnp.concatenate(
        [acc / cnt_a.astype(f32), jnp.zeros((_T, 2, _D), f32)], axis=1)
    orig_a = wpad_ref[:, pl.ds(yl, 8), :]

    cnt_b = y_end - y_start
    lb = y_start - ws
    blk_b = wpad_ref[:, pl.ds(lb, 3 * _BS), :]
    cm_b = lax.broadcasted_iota(jnp.int32, (1, 3 * _BS, 1), 1) == (y - y_start)
    blk_b = jnp.where(tmask & cm_b, mv[:, None, :], blk_b)
    mj_b = (lax.broadcasted_iota(jnp.int32, (1, 3 * _BS, 1), 1)
            < cnt_b).astype(f32)
    mixed_b = (blk_b * mj_b).sum(1) / cnt_b.astype(f32)
    off_b = y - jnp.minimum(y, _N - 8)
    io8 = lax.broadcasted_iota(jnp.int32, (1, 8, 1), 1) == off_b
    mixed_b8 = jnp.where(io8, mixed_b[:, None, :], 0.0)
    orig_b8 = jnp.where(io8, coly[:, None, :], 0.0)

    is_b = y_end > _N - _BS
    mixp_ref[0] = jnp.where(is_b, mixed_b8, mixed_a)
    orig_ref[0] = jnp.where(is_b, orig_b8, orig_a)


_W = 32


def _bt4(c, j):
    bt = c * (_B * _T // 2) + j
    return (bt // _T, bt % _T, 0, 0)


def kernel(input, adapt_G, xff_w, xff_b, ff1_w, ff1_b, ff2_w, ff2_b, ln_w, ln_b):
    f32 = jnp.float32
    B, T, N, D = _B, _T, _N, _D
    BT = B * T
    scale = 1.0 / math.sqrt(D)

    inp3 = input.reshape(BT, N, D)
    gT = adapt_G
    qscale = jnp.concatenate(
        [jnp.full((D,), scale, f32), jnp.ones((2 * D,), f32)])
    xffT_s = xff_w.T * qscale[None, :]
    xffb_s = (xff_b * qscale)[None, :]

    vi_raw, rowstats, y_arr, ws_arr = pl.pallas_call(
        _attn_kernel,
        grid=(2, BT // 2),
        in_specs=[
            pl.BlockSpec((1, N, D), lambda c, j: (c * (BT // 2) + j, 0, 0)),
            pl.BlockSpec((N, N), lambda c, j: (0, 0)),
            pl.BlockSpec((D, 3 * D), lambda c, j: (0, 0)),
            pl.BlockSpec((1, 3 * D), lambda c, j: (0, 0)),
        ],
        out_specs=[
            pl.BlockSpec((1, N, D), lambda c, j: (c * (BT // 2) + j, 0, 0)),
            pl.BlockSpec((1, 2, N), lambda c, j: (c * (BT // 2) + j, 0, 0)),
            pl.BlockSpec((1, 1, 1), lambda c, j: ((c * (BT // 2) + j) // T, 0, 0),
                         memory_space=pltpu.SMEM),
            pl.BlockSpec((1, 1, 1), lambda c, j: ((c * (BT // 2) + j) // T, 0, 0),
                         memory_space=pltpu.SMEM),
        ],
        out_shape=[
            jax.ShapeDtypeStruct((BT, N, D), f32),
            jax.ShapeDtypeStruct((BT, 2, N), f32),
            jax.ShapeDtypeStruct((B, 1, 1), jnp.int32),
            jax.ShapeDtypeStruct((B, 1, 1), jnp.int32),
        ],
        compiler_params=pltpu.CompilerParams(
            dimension_semantics=("parallel", "arbitrary"),
            vmem_limit_bytes=56 * 1024 * 1024,
        ),
        name="attn_fused",
        interpret=_INTERPRET,
    )(inp3, gT, xffT_s, xffb_s)

    y = y_arr.reshape(B)
    ws = ws_arr.reshape(B)
    inp4 = input.reshape(B, T, N, D)
    nb = N // 128
    mixp, orig = pl.pallas_call(
        _win_kernel,
        grid_spec=pltpu.PrefetchScalarGridSpec(
            num_scalar_prefetch=2,
            grid=(B,),
            in_specs=[
                pl.BlockSpec(
                    (1, T, 128, D),
                    lambda b, ws_r, y_r: (b, 0, jnp.minimum(ws_r[b] // 128, nb - 2), 0)),
                pl.BlockSpec(
                    (1, T, 128, D),
                    lambda b, ws_r, y_r: (b, 0, jnp.minimum(ws_r[b] // 128, nb - 2) + 1, 0)),
                pl.BlockSpec((D, D), lambda b, ws_r, y_r: (0, 0)),
                pl.BlockSpec((1, D), lambda b, ws_r, y_r: (0, 0)),
            ],
            out_specs=[
                pl.BlockSpec((1, T, 8, D), lambda b, ws_r, y_r: (b, 0, 0, 0)),
                pl.BlockSpec((1, T, 8, D), lambda b, ws_r, y_r: (b, 0, 0, 0)),
            ],
            scratch_shapes=[pltpu.VMEM((T, 256, D), f32),
                            pltpu.VMEM((T, 2 * _W, D), f32)],
        ),
        out_shape=[jax.ShapeDtypeStruct((B, T, 8, D), f32),
                   jax.ShapeDtypeStruct((B, T, 8, D), f32)],
        name="v_window_mixup",
        interpret=_INTERPRET,
    )(ws, y, inp4, inp4, xffT_s[:, 2 * D:], xffb_s[:, 2 * D:])

    is_b = jnp.minimum(y + 2 * _BS, N) > N - _BS
    cs_arr = jnp.where(is_b, jnp.minimum(y, N - 8), y)
    act = jnp.where(is_b[:, None],
                    jnp.arange(8)[None, :] == (y - cs_arr)[:, None],
                    jnp.arange(8)[None, :] < _BS)
    mix_u8 = lax.stop_gradient(mixp.astype(jnp.uint8).astype(f32))
    delta = jnp.where(act[:, None, :, None]
                      & (jnp.arange(D) < _C_FRONT)[None, None, None, :],
                      mix_u8 - orig, 0.0)
    gcol = jax.vmap(
        lambda c: lax.dynamic_slice(adapt_G, (0, c), (N, 8)))(cs_arr)

    wqkT_s = xffT_s[:, :2 * D]
    bqk_s = xffb_s[:, :2 * D]

    NB = _NB
    ng = BT // NB
    grp = lambda c, j: (c * (ng // 2) + j, 0, 0)
    out = pl.pallas_call(
        _ffn_kernel,
        grid=(2, ng // 2),
        in_specs=[
            pl.BlockSpec((NB, N, D), grp),
            pl.BlockSpec((NB, N, D), grp),
            pl.BlockSpec((NB, 2, N), grp),
            pl.BlockSpec((1, N, 8),
                         lambda c, j: ((c * (ng // 2) + j) * NB // T, 0, 0)),
            pl.BlockSpec((NB, 8, D), grp),
            pl.BlockSpec((D, 2 * D), lambda c, j: (0, 0)),
            pl.BlockSpec((1, 2 * D), lambda c, j: (0, 0)),
            pl.BlockSpec((D, D), lambda c, j: (0, 0)),
            pl.BlockSpec((1, D), lambda c, j: (0, 0)),
            pl.BlockSpec((D, D), lambda c, j: (0, 0)),
            pl.BlockSpec((1, D), lambda c, j: (0, 0)),
            pl.BlockSpec((1, D), lambda c, j: (0, 0)),
            pl.BlockSpec((1, D), lambda c, j: (0, 0)),
        ],
        out_specs=pl.BlockSpec((NB, N, D), grp),
        out_shape=jax.ShapeDtypeStruct((BT, N, D), f32),
        compiler_params=pltpu.CompilerParams(
            dimension_semantics=("parallel", "arbitrary"),
            vmem_limit_bytes=56 * 1024 * 1024,
        ),
        name="mixup_ffn_ln",
        interpret=_INTERPRET,
    )(inp3, vi_raw,
      rowstats, gcol, delta.reshape(BT, 8, D),
      wqkT_s, bqk_s, ff1_w.T, ff1_b[None, :], ff2_w.T, ff2_b[None, :],
      ln_w[None, :], ln_b[None, :])

    return out.reshape(B, T, N, D)
